# baseline (speedup 1.0000x reference)
import functools
import math

import jax
import jax.numpy as jnp
from jax import lax
from jax.experimental import pallas as pl
from jax.experimental.pallas import tpu as pltpu
from jax.experimental.pallas import tpu_sc as plsc

VOC = 32000
N_ROWS = 2048
PAD = 0
SMOOTH = 0.1
CONF = 1.0 - SMOOTH
VALUE = SMOOTH / (VOC - 2)
C1 = (VOC - 2) * VALUE * math.log(VALUE) + CONF * math.log(CONF)

NC = 2
NS = 16
L = 16
NW = NS
BPW = N_ROWS // NW
NB = 32

ROW_BLK = 128
COL_BLK = 32000
GR = N_ROWS // ROW_BLK
GC = VOC // COL_BLK


@functools.partial(
    pl.kernel,
    mesh=plsc.VectorSubcoreMesh(
        core_axis_name="c", subcore_axis_name="s", num_cores=1
    ),
    out_type=jax.ShapeDtypeStruct((NW, L), jnp.float32),
    scratch_types=[
        pltpu.VMEM((BPW,), jnp.int32),
        pltpu.VMEM((NB, 8, 128), jnp.float32),
        pltpu.VMEM((L,), jnp.float32),
        pltpu.SemaphoreType.DMA,
    ],
)
def _sc_gather(x_hbm, tgt_hbm, out_hbm, tgt_v, buf_v, acc_v, sem):
    wid = lax.axis_index("s")
    base = wid * BPW
    pltpu.sync_copy(tgt_hbm.at[pl.ds(base, BPW)], tgt_v)
    lanes = lax.iota(jnp.int32, L)
    acc = jnp.zeros((L,), jnp.float32)
    for b in range(BPW // NB):
        tvs = [tgt_v[pl.ds(b * NB + c * L, L)] for c in range(NB // L)]
        handles = []
        for k in range(NB):
            t = tvs[k // L][k % L]
            row0 = pl.multiple_of(base + b * NB + (k & ~7), 8)
            col0 = pl.multiple_of((t >> 7) << 7, 128)
            handles.append(
                pltpu.async_copy(
                    x_hbm.at[pl.ds(row0, 8), pl.ds(col0, 128)], buf_v.at[k], sem
                )
            )
        for h in handles:
            h.wait()
        for k in range(NB):
            t = tvs[k // L][k % L]
            cb = ((t >> 4) << 4) & 127
            chunk = buf_v[k, k & 7, pl.ds(cb, L)]
            lane_sel = jnp.where(t != PAD, jnp.bitwise_and(t, L - 1), L)
            acc = acc + jnp.where(lanes == lane_sel, chunk, 0.0)
    acc_v[...] = acc
    pltpu.sync_copy(acc_v, out_hbm.at[wid])


def _tc_body(x_ref, t_ref, o_ref):
    r = pl.program_id(0)
    v = pl.program_id(1)

    @pl.when(jnp.logical_and(r == 0, v == 0))
    def _init():
        o_ref[...] = jnp.zeros((1, 1), jnp.float32)

    xb = x_ref[...]
    maskf = (t_ref[...] != PAD).astype(jnp.float32)
    rowsum = jnp.sum(xb, axis=1, keepdims=True)
    acc = -VALUE * jnp.sum(rowsum * maskf)

    @pl.when(v == 0)
    def _col0_and_const():
        x0 = xb[:, 0:1]
        extra = VALUE * jnp.sum(x0 * maskf) + C1 * jnp.sum(maskf)
        o_ref[...] = o_ref[...] + jnp.reshape(extra, (1, 1))

    o_ref[...] = o_ref[...] + jnp.reshape(acc, (1, 1))


def _combine_body(d_ref, p_ref, o_ref):
    o_ref[...] = d_ref[...] + jnp.reshape(
        (VALUE - CONF) * jnp.sum(p_ref[...]), (1, 1)
    )


def kernel(x, target):
    partials = _sc_gather(x, target)
    t2 = target.reshape(N_ROWS, 1)
    dense = pl.pallas_call(
        _tc_body,
        grid=(GR, GC),
        in_specs=[
            pl.BlockSpec((ROW_BLK, COL_BLK), lambda r, v: (r, v)),
            pl.BlockSpec((ROW_BLK, 1), lambda r, v: (r, 0)),
        ],
        out_specs=pl.BlockSpec((1, 1), lambda r, v: (0, 0)),
        out_shape=jax.ShapeDtypeStruct((1, 1), jnp.float32),
    )(x, t2)
    out = pl.pallas_call(
        _combine_body,
        out_shape=jax.ShapeDtypeStruct((1, 1), jnp.float32),
    )(dense, partials)
    return out[0, 0]


# speedup vs baseline: 1.0051x; 1.0051x over previous
import functools
import math

import jax
import jax.numpy as jnp
from jax import lax
from jax.experimental import pallas as pl
from jax.experimental.pallas import tpu as pltpu
from jax.experimental.pallas import tpu_sc as plsc

VOC = 32000
N_ROWS = 2048
PAD = 0
SMOOTH = 0.1
CONF = 1.0 - SMOOTH
VALUE = SMOOTH / (VOC - 2)
C1 = (VOC - 2) * VALUE * math.log(VALUE) + CONF * math.log(CONF)

NC = 2
NS = 16
L = 16
NW = NS
BPW = N_ROWS // NW
NB = 64

ROW_BLK = 256
COL_BLK = 16000
GR = N_ROWS // ROW_BLK
GC = VOC // COL_BLK


@functools.partial(
    pl.kernel,
    mesh=plsc.VectorSubcoreMesh(
        core_axis_name="c", subcore_axis_name="s", num_cores=1
    ),
    out_type=jax.ShapeDtypeStruct((NW, L), jnp.float32),
    scratch_types=[
        pltpu.VMEM((BPW,), jnp.int32),
        pltpu.VMEM((NB, 8, 128), jnp.float32),
        pltpu.VMEM((L,), jnp.float32),
        pltpu.SemaphoreType.DMA,
    ],
)
def _sc_gather(x_hbm, tgt_hbm, out_hbm, tgt_v, buf_v, acc_v, sem):
    wid = lax.axis_index("s")
    base = wid * BPW
    pltpu.sync_copy(tgt_hbm.at[pl.ds(base, BPW)], tgt_v)
    lanes = lax.iota(jnp.int32, L)
    acc = jnp.zeros((L,), jnp.float32)
    for b in range(BPW // NB):
        tvs = [tgt_v[pl.ds(b * NB + c * L, L)] for c in range(NB // L)]
        handles = []
        for k in range(NB):
            t = tvs[k // L][k % L]
            row0 = pl.multiple_of(base + b * NB + (k & ~7), 8)
            col0 = pl.multiple_of((t >> 7) << 7, 128)
            handles.append(
                pltpu.async_copy(
                    x_hbm.at[pl.ds(row0, 8), pl.ds(col0, 128)], buf_v.at[k], sem
                )
            )
        for h in handles:
            h.wait()
        for k in range(NB):
            t = tvs[k // L][k % L]
            cb = ((t >> 4) << 4) & 127
            chunk = buf_v[k, k & 7, pl.ds(cb, L)]
            lane_sel = jnp.where(t != PAD, jnp.bitwise_and(t, L - 1), L)
            acc = acc + jnp.where(lanes == lane_sel, chunk, 0.0)
    acc_v[...] = acc
    pltpu.sync_copy(acc_v, out_hbm.at[wid])


def _tc_body(x_ref, t_ref, o_ref):
    r = pl.program_id(0)
    v = pl.program_id(1)

    @pl.when(jnp.logical_and(r == 0, v == 0))
    def _init():
        o_ref[...] = jnp.zeros((1, 1), jnp.float32)

    xb = x_ref[...]
    maskf = (t_ref[...] != PAD).astype(jnp.float32)
    rowsum = jnp.sum(xb, axis=1, keepdims=True)
    acc = -VALUE * jnp.sum(rowsum * maskf)

    @pl.when(v == 0)
    def _col0_and_const():
        x0 = xb[:, 0:1]
        extra = VALUE * jnp.sum(x0 * maskf) + C1 * jnp.sum(maskf)
        o_ref[...] = o_ref[...] + jnp.reshape(extra, (1, 1))

    o_ref[...] = o_ref[...] + jnp.reshape(acc, (1, 1))


def _combine_body(d_ref, p_ref, o_ref):
    o_ref[...] = d_ref[...] + jnp.reshape(
        (VALUE - CONF) * jnp.sum(p_ref[...]), (1, 1)
    )


def kernel(x, target):
    partials = _sc_gather(x, target)
    t2 = target.reshape(N_ROWS, 1)
    dense = pl.pallas_call(
        _tc_body,
        grid=(GR, GC),
        in_specs=[
            pl.BlockSpec((ROW_BLK, COL_BLK), lambda r, v: (r, v)),
            pl.BlockSpec((ROW_BLK, 1), lambda r, v: (r, 0)),
        ],
        out_specs=pl.BlockSpec((1, 1), lambda r, v: (0, 0)),
        out_shape=jax.ShapeDtypeStruct((1, 1), jnp.float32),
    )(x, t2)
    out = pl.pallas_call(
        _combine_body,
        out_shape=jax.ShapeDtypeStruct((1, 1), jnp.float32),
    )(dense, partials)
    return out[0, 0]
